# baseline (speedup 1.0000x reference)
import functools

import jax
import jax.numpy as jnp
from jax import lax
from jax.experimental import pallas as pl
from jax.experimental.pallas import tpu as pltpu
from jax.experimental.pallas import tpu_sc as plsc

_SIZE = 3
_COLS = 1024
_NW = 32
_CPW = _COLS // _NW
_LANES = 16

_mesh = plsc.VectorSubcoreMesh(core_axis_name="c", subcore_axis_name="s")


@functools.partial(
    pl.kernel,
    mesh=_mesh,
    out_type=jax.ShapeDtypeStruct((_SIZE * _COLS,), jnp.float32),
    scratch_types=[
        pltpu.VMEM((_SIZE * _SIZE * _LANES,), jnp.float32),
        pltpu.VMEM((_SIZE * _CPW,), jnp.float32),
        pltpu.VMEM((_SIZE * _CPW,), jnp.float32),
        pltpu.SemaphoreType.DMA,
    ],
)
def _spmv(v_hbm, m_hbm, out_hbm, m_v, v_v, o_v, sem):
    wid = lax.axis_index("s") * 2 + lax.axis_index("c")
    base = wid * _CPW
    copies = [pltpu.async_copy(m_hbm, m_v, sem)]
    for k in range(_SIZE):
        copies.append(
            pltpu.async_copy(
                v_hbm.at[pl.ds(k * _COLS + base, _CPW)],
                v_v.at[pl.ds(k * _CPW, _CPW)],
                sem,
            )
        )
    for c in copies:
        c.wait()
    for r in range(_SIZE):
        for j in range(_CPW // _LANES):
            acc = None
            for k in range(_SIZE):
                m_rk = m_v[pl.ds((r * _SIZE + k) * _LANES, _LANES)]
                v_kj = v_v[pl.ds(k * _CPW + j * _LANES, _LANES)]
                acc = m_rk * v_kj if acc is None else acc + m_rk * v_kj
            o_v[pl.ds(r * _CPW + j * _LANES, _LANES)] = acc
    out_copies = [
        pltpu.async_copy(
            o_v.at[pl.ds(r * _CPW, _CPW)],
            out_hbm.at[pl.ds(r * _COLS + base, _CPW)],
            sem,
        )
        for r in range(_SIZE)
    ]
    for c in out_copies:
        c.wait()


def kernel(v, M_hat):
    m_b = jnp.broadcast_to(M_hat[:, :, None], (_SIZE, _SIZE, _LANES)).reshape(-1)
    out_flat = _spmv(v.reshape(-1), m_b)
    return out_flat.reshape(_SIZE, _COLS)


# speedup vs baseline: 1.0613x; 1.0613x over previous
import functools

import jax
import jax.numpy as jnp
from jax import lax
from jax.experimental import pallas as pl
from jax.experimental.pallas import tpu as pltpu
from jax.experimental.pallas import tpu_sc as plsc

_SIZE = 3
_COLS = 1024
_N = _SIZE * _COLS
_NW = 16
_EPW = _N // _NW
_LANES = 16

_mesh = plsc.VectorSubcoreMesh(
    core_axis_name="c", subcore_axis_name="s", num_cores=1
)


@functools.partial(
    pl.kernel,
    mesh=_mesh,
    out_type=jax.ShapeDtypeStruct((_N,), jnp.float32),
    scratch_types=[
        pltpu.VMEM((_SIZE * _SIZE * _LANES,), jnp.float32),
        pltpu.VMEM((_N,), jnp.float32),
        pltpu.VMEM((_EPW,), jnp.float32),
        pltpu.SemaphoreType.DMA,
    ],
)
def _spmv(v_hbm, m_hbm, out_hbm, m_v, v_v, o_v, sem):
    wid = lax.axis_index("s")
    base = wid * _EPW
    c_m = pltpu.async_copy(m_hbm, m_v, sem)
    c_v = pltpu.async_copy(v_hbm, v_v, sem)
    c_m.wait()
    c_v.wait()
    for j in range(_EPW // _LANES):
        f0 = base + j * _LANES
        r = lax.shift_right_logical(f0, 10)
        col = f0 - lax.shift_left(r, 10)
        acc = None
        for k in range(_SIZE):
            m_rk = m_v[pl.ds((r * _SIZE + k) * _LANES, _LANES)]
            v_kj = v_v[pl.ds(k * _COLS + col, _LANES)]
            acc = m_rk * v_kj if acc is None else acc + m_rk * v_kj
        o_v[pl.ds(j * _LANES, _LANES)] = acc
    pltpu.async_copy(o_v, out_hbm.at[pl.ds(base, _EPW)], sem).wait()


def kernel(v, M_hat):
    m_b = jnp.broadcast_to(M_hat[:, :, None], (_SIZE, _SIZE, _LANES)).reshape(-1)
    out_flat = _spmv(v.reshape(-1), m_b)
    return out_flat.reshape(_SIZE, _COLS)
